# baseline (speedup 1.0000x reference)
import functools

import jax
import jax.numpy as jnp
from jax import lax
from jax.experimental import pallas as pl
from jax.experimental.pallas import tpu as pltpu
from jax.experimental.pallas import tpu_sc as plsc

B = 4096
L_ITEM = 20
L_CTX = 50
L_CTXP = 64
EMBED = 32
CTXD = 32
NUM_ITEMS = 1000000
SET_ROWS = NUM_ITEMS - 1
SET_TRUNC = SET_ROWS - 3
SET_CLAMP = SET_TRUNC - 1
PACK = 128 // EMBED

NC = 2
NS = 16
NW = NC * NS

BPW = B // NW
ITEM_PW = BPW * L_ITEM
CTX_PW = BPW * L_CTXP
CH = 64
ITEM_CHUNKS = ITEM_PW // CH
CTX_CHUNKS = CTX_PW // CH
IB = 4
EB = 2
CB = 6


@functools.cache
def _sc_gather_fn():
    mesh = plsc.VectorSubcoreMesh(core_axis_name="c", subcore_axis_name="s")

    @functools.partial(
        pl.kernel,
        mesh=mesh,
        out_type=(
            jax.ShapeDtypeStruct((B * L_ITEM, EMBED), jnp.float32),
            jax.ShapeDtypeStruct((B, CTXD), jnp.float32),
        ),
        scratch_types=[
            pltpu.VMEM((ITEM_PW,), jnp.int32),
            pltpu.VMEM((ITEM_PW,), jnp.int32),
            pltpu.VMEM((CTX_PW,), jnp.int32),
            pltpu.VMEM((CTX_PW,), jnp.int32),
            pltpu.VMEM((IB, CH, 128), jnp.float32),
            pltpu.VMEM((EB, CH, EMBED), jnp.float32),
            pltpu.VMEM((CB, CH, 128), jnp.float32),
            pltpu.VMEM((BPW, CTXD), jnp.float32),
            pltpu.SemaphoreType.DMA,
            pltpu.SemaphoreType.DMA,
            pltpu.SemaphoreType.DMA,
        ],
        compiler_params=pltpu.CompilerParams(use_tc_tiling_on_sc=False,
                                             needs_layout_passes=False),
    )
    def _sc_gather(item_idx, ctx_idx, item_sup, set_sup, item_out, sum_out,
                   iq_v, is_v, cq_v, cs_v, isup_bufs, ext_bufs, csup_bufs,
                   acc_v, isem, wsem, csem):
        cid = lax.axis_index("c")
        sid = lax.axis_index("s")
        wid = sid * NC + cid
        iota16 = lax.iota(jnp.int32, 16)

        def bcast_i32(x):
            return jnp.zeros((16,), jnp.int32) + x

        with jax.named_scope("item_adjust"):
            pltpu.sync_copy(item_idx.at[pl.ds(wid * ITEM_PW, ITEM_PW)], iq_v)

            def iadj(g, carry):
                v = iq_v[pl.ds(g * 16, 16)]
                is_v[pl.ds(g * 16, 16)] = (v & 3) << 5
                iq_v[pl.ds(g * 16, 16)] = v >> 2
                return carry

            lax.fori_loop(0, ITEM_PW // 16, iadj, 0)

        def fire_item(j, buf):
            pltpu.async_copy(item_sup.at[iq_v.at[pl.ds(j * CH, CH)]],
                             isup_bufs.at[buf], isem)

        with jax.named_scope("item_prime"):
            for b in range(IB):
                fire_item(b, b)

        with jax.named_scope("ctx_adjust"):
            pltpu.sync_copy(ctx_idx.at[pl.ds(wid * CTX_PW, CTX_PW)], cq_v)

            def cadj(g, carry):
                v = cq_v[pl.ds(g * 16, 16)]
                a = jnp.minimum(jnp.maximum(v - 1, 0), SET_CLAMP)
                cs_v[pl.ds(g * 16, 16)] = (a & 3) << 5
                cq_v[pl.ds(g * 16, 16)] = a >> 2
                return carry

            lax.fori_loop(0, CTX_PW // 16, cadj, 0)

        def fire_ctx(c, buf):
            pltpu.async_copy(set_sup.at[cq_v.at[pl.ds(c * CH, CH)]],
                             csup_bufs.at[buf], csem)

        with jax.named_scope("ctx_prime"):
            for b in range(CB):
                fire_ctx(b, b)

        with jax.named_scope("item_ring"):
            def item_group(g, carry):
                for b in range(IB):
                    j = g * IB + b
                    e = b % EB
                    pltpu.make_async_copy(item_sup.at[pl.ds(0, CH)],
                                          isup_bufs.at[b], isem).wait()

                    @pl.when(j >= EB)
                    def _():
                        pltpu.make_async_copy(
                            ext_bufs.at[0],
                            item_out.at[pl.ds(0, CH)], wsem).wait()

                    sb = isup_bufs.at[b]
                    for r in range(CH):
                        col = plsc.load_gather(
                            is_v, [bcast_i32(j * CH + r)]) + iota16
                        ext_bufs[e, r, pl.ds(0, 16)] = plsc.load_gather(
                            sb, [bcast_i32(r), col])
                        ext_bufs[e, r, pl.ds(16, 16)] = plsc.load_gather(
                            sb, [bcast_i32(r), col + 16])
                    pltpu.async_copy(
                        ext_bufs.at[e],
                        item_out.at[pl.ds(wid * ITEM_PW + j * CH, CH)], wsem)

                    @pl.when(j + IB < ITEM_CHUNKS)
                    def _():
                        fire_item(j + IB, b)
                return carry

            lax.fori_loop(0, ITEM_CHUNKS // IB, item_group, 0)

        with jax.named_scope("ctx_ring"):
            def ctx_group(g, carry):
                for b in range(CB):
                    c = g * CB + b
                    pltpu.make_async_copy(set_sup.at[pl.ds(0, CH)],
                                          csup_bufs.at[b], csem).wait()
                    sb = csup_bufs.at[b]
                    acc0 = jnp.zeros((16,), jnp.float32)
                    acc1 = jnp.zeros((16,), jnp.float32)
                    for r in range(CH):
                        col = plsc.load_gather(
                            cs_v, [bcast_i32(c * CH + r)]) + iota16
                        acc0 = acc0 + plsc.load_gather(
                            sb, [bcast_i32(r), col])
                        acc1 = acc1 + plsc.load_gather(
                            sb, [bcast_i32(r), col + 16])
                    acc_v[c, pl.ds(0, 16)] = acc0
                    acc_v[c, pl.ds(16, 16)] = acc1

                    @pl.when(c + CB < CTX_CHUNKS)
                    def _():
                        fire_ctx(c + CB, b)
                return carry

            lax.fori_loop(0, CTX_CHUNKS // CB, ctx_group, 0)

        with jax.named_scope("tail"):
            pltpu.sync_copy(acc_v, sum_out.at[pl.ds(wid * BPW, BPW)])
            for _ in range(EB):
                pltpu.make_async_copy(
                    ext_bufs.at[0],
                    item_out.at[pl.ds(0, CH)], wsem).wait()

    return _sc_gather


BB = 256


def _tc_body(idx_ref, it_ref, sum_ref, tab_ref, tail_ref,
             w1, b1, w2, b2, w3, b3, out_ref):
    idx = idx_ref[...]
    maskf = (idx > 0).astype(jnp.float32)
    nz = jnp.sum(maskf, axis=-1, keepdims=True)
    tab = tab_ref[...]
    tail = tail_ref[...]
    padsum = jnp.sum(tab[L_CTX - 1:L_CTXP - 1, :], axis=0, keepdims=True)
    summed = (sum_ref[...] - padsum
              - (float(L_CTX) - nz) * tab[0:1, :])
    clamp_off = SET_CLAMP - (SET_ROWS - 16)
    for t in range(SET_TRUNC, SET_ROWS):
        cnt = jnp.sum((idx == t + 1).astype(jnp.float32),
                      axis=-1, keepdims=True)
        off = t - (SET_ROWS - 16)
        summed = summed + cnt * (tail[off:off + 1, :]
                                 - tail[clamp_off:clamp_off + 1, :])
    sq = jnp.sum(summed * summed, axis=-1, keepdims=True)
    normalized = summed * lax.rsqrt(jnp.maximum(sq, 1e-4))
    h = jnp.maximum(
        jnp.dot(normalized, w1[...], preferred_element_type=jnp.float32) + b1[...], 0.0)
    h = jnp.maximum(
        jnp.dot(h, w2[...], preferred_element_type=jnp.float32) + b2[...], 0.0)
    ce = jnp.dot(h, w3[...], preferred_element_type=jnp.float32) + b3[...]
    it = it_ref[...]
    diff = it - ce[:, None, :]
    d = jnp.sqrt(jnp.sum(diff * diff, axis=-1))
    out_ref[...] = 1.0 - jnp.tanh(d)


def _tc_compute(ctx_idx, item_rows, summed, tab64, tail16,
                W1, b1, W2, b2, W3, b3):
    grid = (B // BB,)
    return pl.pallas_call(
        _tc_body,
        grid=grid,
        in_specs=[
            pl.BlockSpec((BB, L_CTX), lambda i: (i, 0)),
            pl.BlockSpec((BB, L_ITEM, EMBED), lambda i: (i, 0, 0)),
            pl.BlockSpec((BB, CTXD), lambda i: (i, 0)),
            pl.BlockSpec((L_CTXP, CTXD), lambda i: (0, 0)),
            pl.BlockSpec((16, CTXD), lambda i: (0, 0)),
            pl.BlockSpec((CTXD, 2 * CTXD), lambda i: (0, 0)),
            pl.BlockSpec((1, 2 * CTXD), lambda i: (0, 0)),
            pl.BlockSpec((2 * CTXD, 4 * CTXD), lambda i: (0, 0)),
            pl.BlockSpec((1, 4 * CTXD), lambda i: (0, 0)),
            pl.BlockSpec((4 * CTXD, EMBED), lambda i: (0, 0)),
            pl.BlockSpec((1, EMBED), lambda i: (0, 0)),
        ],
        out_specs=pl.BlockSpec((BB, L_ITEM), lambda i: (i, 0)),
        out_shape=jax.ShapeDtypeStruct((B, L_ITEM), jnp.float32),
    )(ctx_idx, item_rows, summed, tab64, tail16, W1, b1, W2, b2, W3, b3)


def kernel(item_indices, context_indices, item_table, set_table,
           W1, b1, W2, b2, W3, b3):
    pad_block = jnp.broadcast_to(
        jnp.arange(L_CTX, L_CTXP, dtype=jnp.int32), (B, L_CTXP - L_CTX))
    ctx_pad = jnp.concatenate([context_indices, pad_block], axis=1)
    item_sup = item_table.reshape(NUM_ITEMS // PACK, 128)
    set_sup = set_table[:SET_TRUNC].reshape(SET_TRUNC // PACK, 128)
    item_rows, summed = _sc_gather_fn()(
        item_indices.reshape(-1), ctx_pad.reshape(-1), item_sup, set_sup)
    return _tc_compute(
        context_indices,
        item_rows.reshape(B, L_ITEM, EMBED),
        summed, set_table[:L_CTXP], set_table[SET_ROWS - 16:],
        W1, b1.reshape(1, -1), W2, b2.reshape(1, -1), W3, b3.reshape(1, -1))


# speedup vs baseline: 1.5063x; 1.5063x over previous
import functools

import jax
import jax.numpy as jnp
from jax import lax
from jax.experimental import pallas as pl
from jax.experimental.pallas import tpu as pltpu
from jax.experimental.pallas import tpu_sc as plsc

B = 4096
L_ITEM = 20
L_CTX = 50
L_CTXP = 64
EMBED = 32
CTXD = 32

NC = 2
NS = 16
NW = NC * NS

BPW = B // NW
ITEM_PW = BPW * L_ITEM
CTX_PW = BPW * L_CTXP
CH = 128
ITEM_CHUNKS = ITEM_PW // CH
CTX_CHUNKS = CTX_PW // CH
RPC = CH // L_CTXP
CNB = 16
SDEPTH = 5
GDEPTH = CNB - SDEPTH
IBUF = 8
IWD = 2
IGD = IBUF - IWD


@functools.cache
def _sc_ctx_fn():
    mesh = plsc.VectorSubcoreMesh(core_axis_name="c", subcore_axis_name="s")

    @functools.partial(
        pl.kernel,
        mesh=mesh,
        out_type=jax.ShapeDtypeStruct((B, CTXD), jnp.float32),
        scratch_types=[
            pltpu.VMEM((CTX_PW,), jnp.int32),
            pltpu.VMEM((CNB, CH, CTXD), jnp.float32),
            pltpu.VMEM((CNB, CH), jnp.int32),
            pltpu.VMEM_SHARED((NS * BPW, CTXD), jnp.float32),
            pltpu.SemaphoreType.DMA,
            pltpu.SemaphoreType.DMA,
        ],
        compiler_params=pltpu.CompilerParams(use_tc_tiling_on_sc=False),
    )
    def _sc_ctx(ctx_idx, set_tab, sum_out,
                cidx_v, ctx_bufs, sidx_v, acc_sh, csem, ssem):
        cid = lax.axis_index("c")
        sid = lax.axis_index("s")
        wid = sid * NC + cid

        with jax.named_scope("adjust"):
            pltpu.sync_copy(ctx_idx.at[pl.ds(wid * CTX_PW, CTX_PW)], cidx_v)

            def adjust_body(g, carry):
                v = cidx_v[pl.ds(g * 16, 16)]
                cidx_v[pl.ds(g * 16, 16)] = jnp.maximum(v - 1, 0)
                return carry

            lax.fori_loop(0, CTX_PW // 16, adjust_body, 0)

        with jax.named_scope("zero_acc"):
            def zero_body(r, carry):
                ctx_bufs[0, r, pl.ds(0, 16)] = jnp.zeros((16,), jnp.float32)
                ctx_bufs[0, r, pl.ds(16, 16)] = jnp.zeros((16,), jnp.float32)
                return carry

            lax.fori_loop(0, CH, zero_body, 0)
            pltpu.sync_copy(ctx_bufs.at[0],
                            acc_sh.at[pl.ds(sid * BPW, BPW)])

        def fire_ctx(c, buf):
            pltpu.async_copy(
                set_tab.at[cidx_v.at[pl.ds(c * CH, CH)]],
                ctx_bufs.at[buf], csem)

        with jax.named_scope("ctx_prime"):
            for b in range(GDEPTH):
                fire_ctx(b, b)

        with jax.named_scope("ctx_loop"):
            def ctx_group(g, carry):
                for b in range(CNB):
                    c = g * CNB + b
                    pltpu.make_async_copy(set_tab.at[pl.ds(0, CH)],
                                          ctx_bufs.at[b], csem).wait()
                    base = sid * BPW + RPC * c
                    for t in range(CH // 16):
                        sidx_v[b, pl.ds(t * 16, 16)] = (
                            jnp.zeros((16,), jnp.int32)
                            + (base + (1 if t >= CH // 32 else 0)))
                    pltpu.async_copy(ctx_bufs.at[b], acc_sh.at[sidx_v.at[b]],
                                     ssem, add=True)

                    @pl.when(c >= SDEPTH)
                    def _():
                        pltpu.make_async_copy(
                            ctx_bufs.at[b], acc_sh.at[pl.ds(0, CH)],
                            ssem).wait()

                    @pl.when(c + GDEPTH < CTX_CHUNKS)
                    def _():
                        fire_ctx(c + GDEPTH, (b + GDEPTH) % CNB)
                return carry

            lax.fori_loop(0, CTX_CHUNKS // CNB, ctx_group, 0)

        with jax.named_scope("tail"):
            for _ in range(SDEPTH):
                pltpu.make_async_copy(ctx_bufs.at[0], acc_sh.at[pl.ds(0, CH)],
                                      ssem).wait()
            pltpu.sync_copy(acc_sh.at[pl.ds(sid * BPW, BPW)],
                            sum_out.at[pl.ds(wid * BPW, BPW)])

    return _sc_ctx


@functools.cache
def _sc_item_fn():
    mesh = plsc.VectorSubcoreMesh(core_axis_name="c", subcore_axis_name="s")

    @functools.partial(
        pl.kernel,
        mesh=mesh,
        out_type=jax.ShapeDtypeStruct((B * L_ITEM, EMBED), jnp.float32),
        scratch_types=[
            pltpu.VMEM((ITEM_PW,), jnp.int32),
            pltpu.VMEM((ITEM_CHUNKS, CH, EMBED), jnp.float32),
            pltpu.SemaphoreType.DMA,
            pltpu.SemaphoreType.DMA,
        ],
        compiler_params=pltpu.CompilerParams(use_tc_tiling_on_sc=False),
    )
    def _sc_item(item_idx, item_tab, item_out, iidx_v, item_bufs, isem, wsem):
        cid = lax.axis_index("c")
        sid = lax.axis_index("s")
        wid = sid * NC + cid

        with jax.named_scope("item_fire"):
            pltpu.sync_copy(item_idx.at[pl.ds(wid * ITEM_PW, ITEM_PW)], iidx_v)
            for j in range(ITEM_CHUNKS):
                pltpu.async_copy(
                    item_tab.at[iidx_v.at[pl.ds(j * CH, CH)]],
                    item_bufs.at[j], isem)

        with jax.named_scope("item_drain"):
            for j in range(ITEM_CHUNKS):
                pltpu.make_async_copy(item_tab.at[pl.ds(0, CH)],
                                      item_bufs.at[j], isem).wait()
                pltpu.async_copy(
                    item_bufs.at[j],
                    item_out.at[pl.ds(wid * ITEM_PW + j * CH, CH)], wsem)
            for j in range(ITEM_CHUNKS):
                pltpu.make_async_copy(
                    item_bufs.at[j],
                    item_out.at[pl.ds(wid * ITEM_PW + j * CH, CH)],
                    wsem).wait()

    return _sc_item


BB = 256


def _tc_body(idx_ref, it_ref, sum_ref, tab_ref, w1, b1, w2, b2, w3, b3,
             out_ref):
    maskf = (idx_ref[...] > 0).astype(jnp.float32)
    nz = jnp.sum(maskf, axis=-1, keepdims=True)
    tab = tab_ref[...]
    padsum = jnp.sum(tab[L_CTX - 1:L_CTXP - 1, :], axis=0, keepdims=True)
    summed = (sum_ref[...] - padsum
              - (float(L_CTX) - nz) * tab[0:1, :])
    sq = jnp.sum(summed * summed, axis=-1, keepdims=True)
    normalized = summed * lax.rsqrt(jnp.maximum(sq, 1e-4))
    h = jnp.maximum(
        jnp.dot(normalized, w1[...], preferred_element_type=jnp.float32) + b1[...], 0.0)
    h = jnp.maximum(
        jnp.dot(h, w2[...], preferred_element_type=jnp.float32) + b2[...], 0.0)
    ce = jnp.dot(h, w3[...], preferred_element_type=jnp.float32) + b3[...]
    it = it_ref[...]
    diff = it - ce[:, None, :]
    d = jnp.sqrt(jnp.sum(diff * diff, axis=-1))
    out_ref[...] = 1.0 - jnp.tanh(d)


def _tc_compute(ctx_idx, item_rows, summed, tab64, W1, b1, W2, b2, W3, b3):
    grid = (B // BB,)
    return pl.pallas_call(
        _tc_body,
        grid=grid,
        in_specs=[
            pl.BlockSpec((BB, L_CTX), lambda i: (i, 0)),
            pl.BlockSpec((BB, L_ITEM, EMBED), lambda i: (i, 0, 0)),
            pl.BlockSpec((BB, CTXD), lambda i: (i, 0)),
            pl.BlockSpec((L_CTXP, CTXD), lambda i: (0, 0)),
            pl.BlockSpec((CTXD, 2 * CTXD), lambda i: (0, 0)),
            pl.BlockSpec((1, 2 * CTXD), lambda i: (0, 0)),
            pl.BlockSpec((2 * CTXD, 4 * CTXD), lambda i: (0, 0)),
            pl.BlockSpec((1, 4 * CTXD), lambda i: (0, 0)),
            pl.BlockSpec((4 * CTXD, EMBED), lambda i: (0, 0)),
            pl.BlockSpec((1, EMBED), lambda i: (0, 0)),
        ],
        out_specs=pl.BlockSpec((BB, L_ITEM), lambda i: (i, 0)),
        out_shape=jax.ShapeDtypeStruct((B, L_ITEM), jnp.float32),
    )(ctx_idx, item_rows, summed, tab64, W1, b1, W2, b2, W3, b3)


def kernel(item_indices, context_indices, item_table, set_table,
           W1, b1, W2, b2, W3, b3):
    pad_block = jnp.broadcast_to(
        jnp.arange(L_CTX, L_CTXP, dtype=jnp.int32), (B, L_CTXP - L_CTX))
    ctx_pad = jnp.concatenate([context_indices, pad_block], axis=1)
    summed = _sc_ctx_fn()(ctx_pad.reshape(-1), set_table)
    item_rows = _sc_item_fn()(item_indices.reshape(-1), item_table)
    return _tc_compute(
        context_indices,
        item_rows.reshape(B, L_ITEM, EMBED),
        summed, set_table[:L_CTXP],
        W1, b1.reshape(1, -1), W2, b2.reshape(1, -1), W3, b3.reshape(1, -1))
